# baseline (speedup 1.0000x reference)
import jax
import jax.numpy as jnp
from jax.experimental import pallas as pl

B, NBOX, NCLS, NT = 32, 20000, 81, 50
IOU_THRESH = 0.5
NTP = 56
BN = 5120


def _ssd_block(db_ref, t_ref, mb_ref, lc_ref):
    t = t_ref[0]
    tx0 = t[:, 0:1]
    ty0 = t[:, 1:2]
    tx1 = t[:, 2:3]
    ty1 = t[:, 3:4]
    area_t = (tx1 - tx0) * (ty1 - ty0)

    db = db_ref[...]
    hw = db[2:3, :] * 0.5
    hh = db[3:4, :] * 0.5
    dx0 = db[0:1, :] - hw
    dx1 = db[0:1, :] + hw
    dy0 = db[1:2, :] - hh
    dy1 = db[1:2, :] + hh
    area_d = (dx1 - dx0) * (dy1 - dy0)

    iw = jnp.clip(jnp.minimum(dx1, tx1) - jnp.maximum(dx0, tx0), 0.0)
    ih = jnp.clip(jnp.minimum(dy1, ty1) - jnp.maximum(dy0, ty0), 0.0)
    inter = iw * ih
    g = inter / jnp.maximum(area_d + area_t, 1e-9)

    best_g = jnp.max(g, axis=0, keepdims=True)
    row = jax.lax.broadcasted_iota(jnp.int32, g.shape, 0)
    best = jnp.min(jnp.where(g == best_g, row, NTP),
                   axis=0, keepdims=True)
    onehot = (row == best).astype(jnp.bfloat16)
    keep = (best_g >= 1.0 / 3.0).astype(jnp.float32)

    t_hi = t.astype(jnp.bfloat16)
    r1 = t - t_hi.astype(jnp.float32)
    t_mid = r1.astype(jnp.bfloat16)
    t_lo = (r1 - t_mid.astype(jnp.float32)).astype(jnp.bfloat16)
    dn = (((0,), (0,)), ((), ()))

    def dot(a):
        return jax.lax.dot_general(a, onehot, dn,
                                   preferred_element_type=jnp.float32)

    m1 = dot(t_hi) + dot(t_mid) + dot(t_lo)
    cls_row = m1[4:5, :] * keep
    sel = jnp.concatenate([m1[0:4, :], cls_row, m1[5:8, :]], axis=0)
    lc_ref[0] = sel
    cls_i = cls_row.astype(jnp.int32)
    cls_col = jnp.transpose(cls_i)
    ccol = jax.lax.broadcasted_iota(jnp.int32, (cls_col.shape[0], NCLS), 1)
    mb_ref[0] = (ccol == cls_col).astype(jnp.int32)


@jax.jit
def kernel(loc, conf, defaultbox, target):
    del loc, conf
    dbT = jnp.transpose(defaultbox)
    tp = jnp.pad(target, ((0, 0), (0, NTP - NT), (0, 8 - target.shape[-1])))

    grid = (B, pl.cdiv(NBOX, BN))
    matchbox, loc_conf = pl.pallas_call(
        _ssd_block,
        grid=grid,
        in_specs=[
            pl.BlockSpec((4, BN), lambda i, j: (0, j)),
            pl.BlockSpec((1, NTP, 8), lambda i, j: (i, 0, 0)),
        ],
        out_specs=[
            pl.BlockSpec((1, BN, NCLS), lambda i, j: (i, j, 0)),
            pl.BlockSpec((1, 8, BN), lambda i, j: (i, 0, j)),
        ],
        out_shape=[
            jax.ShapeDtypeStruct((B, NBOX, NCLS), jnp.int32),
            jax.ShapeDtypeStruct((B, 8, NBOX), jnp.float32),
        ],
    )(dbT, tp)
    return matchbox, jnp.transpose(loc_conf[:, 0:4, :], (0, 2, 1))


# speedup vs baseline: 1.0928x; 1.0928x over previous
import jax
import jax.numpy as jnp
from jax.experimental import pallas as pl

B, NBOX, NCLS, NT = 32, 20000, 81, 50
IOU_THRESH = 0.5
NTP = 56
BN = 5120


def _ssd_block(db_ref, t_ref, mb_ref, lc_ref):
    t = t_ref[0]
    tx0 = t[:, 0:1]
    ty0 = t[:, 1:2]
    tx1 = t[:, 2:3]
    ty1 = t[:, 3:4]
    area_t = (tx1 - tx0) * (ty1 - ty0)

    db = db_ref[...]
    hw = db[2:3, :] * 0.5
    hh = db[3:4, :] * 0.5
    dx0 = db[0:1, :] - hw
    dx1 = db[0:1, :] + hw
    dy0 = db[1:2, :] - hh
    dy1 = db[1:2, :] + hh
    area_d = (dx1 - dx0) * (dy1 - dy0)

    iw = jnp.clip(jnp.minimum(dx1, tx1) - jnp.maximum(dx0, tx0), 0.0)
    ih = jnp.clip(jnp.minimum(dy1, ty1) - jnp.maximum(dy0, ty0), 0.0)
    inter = iw * ih
    g = inter / jnp.maximum(area_d + area_t, 1e-9)

    best_g = jnp.max(g, axis=0, keepdims=True)
    row = jax.lax.broadcasted_iota(jnp.int32, g.shape, 0)
    best = jnp.min(jnp.where(g == best_g, row, NTP),
                   axis=0, keepdims=True)
    onehot = (row == best).astype(jnp.bfloat16)
    keep = (best_g >= 1.0 / 3.0).astype(jnp.float32)

    t_hi = t.astype(jnp.bfloat16)
    r1 = t - t_hi.astype(jnp.float32)
    t_mid = r1.astype(jnp.bfloat16)
    t_lo = (r1 - t_mid.astype(jnp.float32)).astype(jnp.bfloat16)
    dn = (((0,), (0,)), ((), ()))

    def dot(a):
        return jax.lax.dot_general(a, onehot, dn,
                                   preferred_element_type=jnp.float32)

    m1 = dot(t_hi) + dot(t_mid) + dot(t_lo)
    cls_row = m1[4:5, :] * keep
    lc_ref[0] = m1[0:4, :]
    cls_i = cls_row.astype(jnp.int8)
    cls_col = jnp.transpose(cls_i)
    ccol = jax.lax.broadcasted_iota(jnp.int8, (cls_col.shape[0], NCLS), 1)
    mb_ref[0] = (ccol == cls_col).astype(jnp.int8)


@jax.jit
def kernel(loc, conf, defaultbox, target):
    del loc, conf
    dbT = jnp.transpose(defaultbox)
    tp = jnp.pad(target, ((0, 0), (0, NTP - NT), (0, 8 - target.shape[-1])))

    grid = (B, pl.cdiv(NBOX, BN))
    matchbox, loc_conf = pl.pallas_call(
        _ssd_block,
        grid=grid,
        in_specs=[
            pl.BlockSpec((4, BN), lambda i, j: (0, j)),
            pl.BlockSpec((1, NTP, 8), lambda i, j: (i, 0, 0)),
        ],
        out_specs=[
            pl.BlockSpec((1, BN, NCLS), lambda i, j: (i, j, 0)),
            pl.BlockSpec((1, 4, BN), lambda i, j: (i, 0, j)),
        ],
        out_shape=[
            jax.ShapeDtypeStruct((B, NBOX, NCLS), jnp.int8),
            jax.ShapeDtypeStruct((B, 4, NBOX), jnp.float32),
        ],
    )(dbT, tp)
    return matchbox.astype(jnp.int32), jnp.transpose(loc_conf, (0, 2, 1))


# speedup vs baseline: 1.1940x; 1.0926x over previous
import jax
import jax.numpy as jnp
from jax.experimental import pallas as pl

B, NBOX, NCLS, NT = 32, 20000, 81, 50
IOU_THRESH = 0.5
NTP = 56
BN = 20096


def _ssd_block(db_ref, t_ref, mb_ref, lc_ref):
    t = t_ref[0]
    tx0 = t[:, 0:1]
    ty0 = t[:, 1:2]
    tx1 = t[:, 2:3]
    ty1 = t[:, 3:4]
    area_t = (tx1 - tx0) * (ty1 - ty0)

    db = db_ref[...]
    hw = db[2:3, :] * 0.5
    hh = db[3:4, :] * 0.5
    dx0 = db[0:1, :] - hw
    dx1 = db[0:1, :] + hw
    dy0 = db[1:2, :] - hh
    dy1 = db[1:2, :] + hh
    area_d = (dx1 - dx0) * (dy1 - dy0)

    iw = jnp.clip(jnp.minimum(dx1, tx1) - jnp.maximum(dx0, tx0), 0.0)
    ih = jnp.clip(jnp.minimum(dy1, ty1) - jnp.maximum(dy0, ty0), 0.0)
    inter = iw * ih
    g = inter / jnp.maximum(area_d + area_t, 1e-9)

    best_g = jnp.max(g, axis=0, keepdims=True)
    row = jax.lax.broadcasted_iota(jnp.int32, g.shape, 0)
    best = jnp.min(jnp.where(g == best_g, row, NTP),
                   axis=0, keepdims=True)
    onehot = (row == best).astype(jnp.bfloat16)
    keep = (best_g >= 1.0 / 3.0).astype(jnp.float32)

    t_hi = t.astype(jnp.bfloat16)
    r1 = t - t_hi.astype(jnp.float32)
    t_mid = r1.astype(jnp.bfloat16)
    t_lo = (r1 - t_mid.astype(jnp.float32)).astype(jnp.bfloat16)
    dn = (((0,), (0,)), ((), ()))

    def dot(a):
        return jax.lax.dot_general(a, onehot, dn,
                                   preferred_element_type=jnp.float32)

    m1 = dot(t_hi) + dot(t_mid) + dot(t_lo)
    cls_row = m1[4:5, :] * keep
    lc_ref[0] = m1[0:4, :]
    cls_i = cls_row.astype(jnp.int8)
    cls_col = jnp.transpose(cls_i)
    ccol = jax.lax.broadcasted_iota(jnp.int8, (cls_col.shape[0], NCLS), 1)
    mb_ref[0] = (ccol == cls_col).astype(jnp.int8)


@jax.jit
def kernel(loc, conf, defaultbox, target):
    del loc, conf
    dbT = jnp.transpose(defaultbox)
    tp = jnp.pad(target, ((0, 0), (0, NTP - NT), (0, 8 - target.shape[-1])))

    grid = (B, pl.cdiv(NBOX, BN))
    matchbox, loc_conf = pl.pallas_call(
        _ssd_block,
        grid=grid,
        in_specs=[
            pl.BlockSpec((4, BN), lambda i, j: (0, j)),
            pl.BlockSpec((1, NTP, 8), lambda i, j: (i, 0, 0)),
        ],
        out_specs=[
            pl.BlockSpec((1, BN, NCLS), lambda i, j: (i, j, 0)),
            pl.BlockSpec((1, 4, BN), lambda i, j: (i, 0, j)),
        ],
        out_shape=[
            jax.ShapeDtypeStruct((B, NBOX, NCLS), jnp.int8),
            jax.ShapeDtypeStruct((B, 4, NBOX), jnp.float32),
        ],
    )(dbT, tp)
    return matchbox.astype(jnp.int32), jnp.transpose(loc_conf, (0, 2, 1))


# speedup vs baseline: 1.1941x; 1.0001x over previous
import jax
import jax.numpy as jnp
from jax.experimental import pallas as pl

B, NBOX, NCLS, NT = 32, 20000, 81, 50
IOU_THRESH = 0.5
NTP = 56
BN = 20096


def _ssd_block(db_ref, t_ref, mb_ref, lc_ref):
    t = t_ref[0]
    tx0 = t[:, 0:1]
    ty0 = t[:, 1:2]
    tx1 = t[:, 2:3]
    ty1 = t[:, 3:4]
    area_t = (tx1 - tx0) * (ty1 - ty0)

    db = db_ref[...]
    hw = db[2:3, :] * 0.5
    hh = db[3:4, :] * 0.5
    dx0 = db[0:1, :] - hw
    dx1 = db[0:1, :] + hw
    dy0 = db[1:2, :] - hh
    dy1 = db[1:2, :] + hh
    area_d = (dx1 - dx0) * (dy1 - dy0)

    iw = jnp.clip(jnp.minimum(dx1, tx1) - jnp.maximum(dx0, tx0), 0.0)
    ih = jnp.clip(jnp.minimum(dy1, ty1) - jnp.maximum(dy0, ty0), 0.0)
    inter = iw * ih
    g = inter / jnp.maximum(area_d + area_t, 1e-9)

    best_g = jnp.max(g, axis=0, keepdims=True)
    row = jax.lax.broadcasted_iota(jnp.int32, g.shape, 0)
    best = jnp.min(jnp.where(g == best_g, row, NTP),
                   axis=0, keepdims=True)
    onehot = (row == best).astype(jnp.bfloat16)
    keep = (best_g >= 1.0 / 3.0).astype(jnp.float32)

    t_hi = t.astype(jnp.bfloat16)
    r1 = t - t_hi.astype(jnp.float32)
    t_mid = r1.astype(jnp.bfloat16)
    t_lo = (r1 - t_mid.astype(jnp.float32)).astype(jnp.bfloat16)
    dn = (((0,), (0,)), ((), ()))

    def dot(a):
        return jax.lax.dot_general(a, onehot, dn,
                                   preferred_element_type=jnp.float32)

    m1 = dot(t_hi) + dot(t_mid) + dot(t_lo)
    cls_row = m1[4:5, :] * keep
    lc_ref[0] = m1[0:4, :]
    cls_i = cls_row.astype(jnp.int8)
    cls_col = jnp.transpose(cls_i)
    ccol = jax.lax.broadcasted_iota(jnp.int8, (cls_col.shape[0], 128), 1)
    mb_ref[0] = (ccol == cls_col).astype(jnp.int8)


@jax.jit
def kernel(loc, conf, defaultbox, target):
    del loc, conf
    dbT = jnp.transpose(defaultbox)
    tp = jnp.pad(target, ((0, 0), (0, NTP - NT), (0, 8 - target.shape[-1])))

    grid = (B, pl.cdiv(NBOX, BN))
    matchbox, loc_conf = pl.pallas_call(
        _ssd_block,
        grid=grid,
        in_specs=[
            pl.BlockSpec((4, BN), lambda i, j: (0, j)),
            pl.BlockSpec((1, NTP, 8), lambda i, j: (i, 0, 0)),
        ],
        out_specs=[
            pl.BlockSpec((1, BN, 128), lambda i, j: (i, j, 0)),
            pl.BlockSpec((1, 4, BN), lambda i, j: (i, 0, j)),
        ],
        out_shape=[
            jax.ShapeDtypeStruct((B, NBOX, 128), jnp.int8),
            jax.ShapeDtypeStruct((B, 4, NBOX), jnp.float32),
        ],
    )(dbT, tp)
    return matchbox[:, :, :NCLS].astype(jnp.int32), jnp.transpose(loc_conf, (0, 2, 1))
